# baseline (speedup 1.0000x reference)
import functools

import jax
import jax.numpy as jnp
from jax import lax
from jax.experimental import pallas as pl
from jax.experimental.pallas import tpu as pltpu
from jax.experimental.pallas import tpu_sc as plsc

N = 10000
NPAD = 10112
E = 320000
EPAD = 327680
NC, NS = 2, 16
CHUNK = 128
DSTEPS = EPAD // (NC * NS * CHUNK)
STEPS = EPAD // (NS * CHUNK)
RPT = NPAD // NS
HID = 64
HW = HID // 2
OUT = 32
DEGW = 16

_mesh = plsc.VectorSubcoreMesh(core_axis_name="c", subcore_axis_name="s")


def _zero_rows(zbuf, dst_sh, r0):
    for k in range(4):
        pltpu.sync_copy(zbuf, dst_sh.at[pl.ds(r0 + k * CHUNK, CHUNK)])
    pltpu.sync_copy(zbuf.at[pl.ds(0, RPT - 4 * CHUNK)],
                    dst_sh.at[pl.ds(r0 + 4 * CHUNK, RPT - 4 * CHUNK)])


def _fill(buf, rows, value):
    def body(i, carry):
        buf[i] = jnp.full((buf.shape[1],), value, jnp.float32)
        return carry
    lax.fori_loop(0, rows, body, 0)


@functools.partial(
    pl.kernel,
    out_type=jax.ShapeDtypeStruct((NC, NPAD, DEGW), jnp.float32),
    mesh=_mesh,
    scratch_types=[
        pltpu.VMEM((DSTEPS, CHUNK), jnp.int32),
        pltpu.VMEM((CHUNK, DEGW), jnp.float32),
        pltpu.VMEM((CHUNK, DEGW), jnp.float32),
        pltpu.VMEM_SHARED((NPAD, DEGW), jnp.float32),
    ],
    compiler_params=pltpu.CompilerParams(use_tc_tiling_on_sc=False),
)
def _deg_pass(dst_hbm, out_hbm, dst_v, ones_v, zbuf, deg_sh):
    c = lax.axis_index("c")
    s = lax.axis_index("s")
    r0 = pl.multiple_of(s * RPT, 8)
    pltpu.sync_copy(dst_hbm.at[c, s], dst_v)
    _fill(zbuf, CHUNK, 0.0)
    _fill(ones_v, CHUNK, 1.0)
    _zero_rows(zbuf, deg_sh, r0)
    plsc.subcore_barrier()

    def step(j, carry):
        pltpu.sync_copy(ones_v, deg_sh.at[dst_v.at[j]], add=True)
        return carry
    lax.fori_loop(0, DSTEPS, step, 0)

    plsc.subcore_barrier()
    pltpu.sync_copy(deg_sh.at[pl.ds(r0, RPT)], out_hbm.at[c, pl.ds(r0, RPT)])


NBUF = 8
HALF = NBUF // 2
GROUPS = STEPS // NBUF


@functools.partial(
    pl.kernel,
    out_type=(jax.ShapeDtypeStruct((NC, NPAD, HW), jnp.float32),
              jax.ShapeDtypeStruct((NC, NPAD, HW), jnp.float32)),
    mesh=_mesh,
    scratch_types=[
        pltpu.VMEM((STEPS, CHUNK), jnp.int32),
        pltpu.VMEM((STEPS, CHUNK), jnp.int32),
        pltpu.VMEM((NBUF, CHUNK, HW), jnp.float32),
        pltpu.VMEM((CHUNK, HW), jnp.float32),
        pltpu.VMEM((CHUNK, HW), jnp.float32),
        pltpu.VMEM((CHUNK, HW), jnp.float32),
        pltpu.VMEM((HW,), jnp.float32),
        pltpu.VMEM_SHARED((NPAD, HW), jnp.float32),
        pltpu.VMEM_SHARED((NPAD, HW), jnp.float32),
        pltpu.SemaphoreType.DMA((NBUF,)),
        pltpu.SemaphoreType.DMA((NBUF,)),
    ],
    compiler_params=pltpu.CompilerParams(use_tc_tiling_on_sc=False),
)
def _gcn_core(tab_hbm, dinvw_hbm, b1_hbm, src_hbm, dst_hbm,
              acc2_hbm, h2s_hbm, src_v, dst_v, rows_v, zbuf,
              ew_a, ew_d, b1_v, acc_sh, tab_sh, gsem, ssem):
    c = lax.axis_index("c")
    s = lax.axis_index("s")
    r0 = pl.multiple_of(s * RPT, 8)
    pltpu.sync_copy(src_hbm.at[s], src_v)
    pltpu.sync_copy(dst_hbm.at[s], dst_v)
    pltpu.sync_copy(b1_hbm.at[c], b1_v)
    _fill(zbuf, CHUNK, 0.0)
    _zero_rows(zbuf, acc_sh, r0)
    pltpu.sync_copy(tab_hbm.at[c, pl.ds(r0, RPT)], tab_sh.at[pl.ds(r0, RPT)])
    plsc.subcore_barrier()

    def start_gather(j, b):
        pltpu.async_copy(tab_sh.at[src_v.at[j]], rows_v.at[b], gsem.at[b])

    def wait_gather(j, b):
        pltpu.make_async_copy(
            tab_sh.at[src_v.at[j]], rows_v.at[b], gsem.at[b]).wait()

    def start_scatter(j, b):
        pltpu.async_copy(rows_v.at[b], acc_sh.at[dst_v.at[j]], ssem.at[b],
                         add=True)

    def wait_scatter(j, b):
        pltpu.make_async_copy(
            rows_v.at[b], acc_sh.at[dst_v.at[j]], ssem.at[b]).wait()

    def edge_pass():
        for b in range(HALF):
            start_gather(b, b)

        def group(g, carry):
            for b in range(NBUF):
                j = g * NBUF + b
                wait_gather(j, b)
                start_scatter(j, b)
                bn = (b + HALF) % NBUF
                if b < HALF:
                    @pl.when(g >= 1)
                    def _():
                        wait_scatter((g - 1) * NBUF + bn, bn)
                    start_gather(j + HALF, bn)
                else:
                    @pl.when(g < GROUPS - 1)
                    def _():
                        wait_scatter(g * NBUF + bn, bn)
                        start_gather(j + HALF, bn)
            return carry
        lax.fori_loop(0, GROUPS, group, 0)

        for b in range(NBUF):
            wait_scatter(STEPS - NBUF + b, b)

    edge_pass()
    plsc.subcore_barrier()

    b1a = b1_v[pl.ds(0, 16)]
    b1b = b1_v[pl.ds(16, 16)]
    ew_t = rows_v.at[0]
    for k in range(5):
        rk = CHUNK if k < 4 else RPT - 4 * CHUNK
        base = r0 + k * CHUNK
        pltpu.sync_copy(acc_sh.at[pl.ds(base, rk)], ew_a.at[pl.ds(0, rk)])
        pltpu.sync_copy(dinvw_hbm.at[pl.ds(base, rk)], ew_d.at[pl.ds(0, rk)])
        pltpu.sync_copy(tab_sh.at[pl.ds(base, rk)], ew_t.at[pl.ds(0, rk)])

        def ew(i, carry):
            for t, b1t in ((0, b1a), (1, b1b)):
                sl = pl.ds(t * 16, 16)
                dv = ew_d[i, sl]
                agg = dv * (ew_a[i, sl] + ew_t[i, sl])
                ew_a[i, sl] = dv * jnp.maximum(agg + b1t, 0.0)
            return carry
        lax.fori_loop(0, rk, ew, 0)
        pltpu.sync_copy(ew_a.at[pl.ds(0, rk)],
                        h2s_hbm.at[c, pl.ds(base, rk)])
        pltpu.sync_copy(zbuf.at[pl.ds(0, rk)], acc_sh.at[pl.ds(base, rk)])
    pltpu.sync_copy(h2s_hbm.at[c, pl.ds(r0, RPT)], tab_sh.at[pl.ds(r0, RPT)])
    plsc.subcore_barrier()

    edge_pass()
    plsc.subcore_barrier()
    pltpu.sync_copy(acc_sh.at[pl.ds(r0, RPT)], acc2_hbm.at[c, pl.ds(r0, RPT)])


def _dinv(degp_ref):
    deg = degp_ref[0, 0:N, 0:1] + degp_ref[1, 0:N, 0:1] + 1.0
    return lax.rsqrt(deg)


def _tc_b_body(x_ref, w1_ref, degp_ref, out_ref, dinvw_ref):
    dinv = _dinv(degp_ref)
    h = jnp.dot(x_ref[...], w1_ref[...], preferred_element_type=jnp.float32)
    out_ref[0, 0:N, :] = dinv * h[:, 0:HW]
    out_ref[1, 0:N, :] = dinv * h[:, HW:HID]
    dinvw_ref[0:N, :] = jnp.broadcast_to(dinv, (N, HW))


def _tc_f_body(a_ref, h2s_ref, degp_ref, w2_ref, b2_ref, w3_ref, b3_ref,
               mu_ref, lv_ref):
    dinv = _dinv(degp_ref)
    aggl = dinv * (a_ref[0, 0:N] + h2s_ref[0, 0:N])
    aggr = dinv * (a_ref[1, 0:N] + h2s_ref[1, 0:N])
    mu_ref[...] = (
        jnp.dot(aggl, w2_ref[0:HW, :], preferred_element_type=jnp.float32)
        + jnp.dot(aggr, w2_ref[HW:HID, :], preferred_element_type=jnp.float32)
        + b2_ref[...])
    lv_ref[...] = (
        jnp.dot(aggl, w3_ref[0:HW, :], preferred_element_type=jnp.float32)
        + jnp.dot(aggr, w3_ref[HW:HID, :], preferred_element_type=jnp.float32)
        + b3_ref[...])


_TC_PARAMS = pltpu.CompilerParams(vmem_limit_bytes=100 * 1024 * 1024)
_tc_b = pl.pallas_call(
    _tc_b_body,
    out_shape=(jax.ShapeDtypeStruct((NC, NPAD, HW), jnp.float32),
               jax.ShapeDtypeStruct((NPAD, HW), jnp.float32)),
    compiler_params=_TC_PARAMS)
_tc_f = pl.pallas_call(
    _tc_f_body,
    out_shape=(jax.ShapeDtypeStruct((N, OUT), jnp.float32),
               jax.ShapeDtypeStruct((N, OUT), jnp.float32)),
    compiler_params=_TC_PARAMS)


@jax.jit
def kernel(x, edge_index, W1, b1, W2, b2, W3, b3):
    ei = edge_index.astype(jnp.int32)
    pad = jnp.full((2, EPAD - E), N, jnp.int32)
    eip = jnp.concatenate([ei, pad], axis=1)
    src2, dst2 = eip[0], eip[1]
    srcd = src2.reshape(NC, NS, DSTEPS, CHUNK)
    dstd = dst2.reshape(NC, NS, DSTEPS, CHUNK)
    srcp = src2.reshape(NS, STEPS, CHUNK)
    dstp = dst2.reshape(NS, STEPS, CHUNK)

    degp = _deg_pass(dstd)
    h1s, dinvw = _tc_b(x, W1, degp)
    acc2, h2s = _gcn_core(h1s, dinvw, b1.reshape(NC, HW), srcp, dstp)
    mu, logvar = _tc_f(acc2, h2s, degp,
                       W2, b2.reshape(1, OUT), W3, b3.reshape(1, OUT))
    return (mu, logvar)


# speedup vs baseline: 1.0066x; 1.0066x over previous
import functools

import jax
import jax.numpy as jnp
from jax import lax
from jax.experimental import pallas as pl
from jax.experimental.pallas import tpu as pltpu
from jax.experimental.pallas import tpu_sc as plsc

N = 10000
NPAD = 10112
E = 320000
EPAD = 327680
NC, NS = 2, 16
CHUNK = 128
DSTEPS = EPAD // (NC * NS * CHUNK)
STEPS = EPAD // (NS * CHUNK)
RPT = NPAD // NS
HID = 64
HW = HID // 2
OUT = 32
DEGW = 16

_mesh = plsc.VectorSubcoreMesh(core_axis_name="c", subcore_axis_name="s")


def _zero_rows(zbuf, dst_sh, r0):
    for k in range(4):
        pltpu.sync_copy(zbuf, dst_sh.at[pl.ds(r0 + k * CHUNK, CHUNK)])
    pltpu.sync_copy(zbuf.at[pl.ds(0, RPT - 4 * CHUNK)],
                    dst_sh.at[pl.ds(r0 + 4 * CHUNK, RPT - 4 * CHUNK)])


def _fill(buf, rows, value):
    def body(i, carry):
        buf[i] = jnp.full((buf.shape[1],), value, jnp.float32)
        return carry
    lax.fori_loop(0, rows, body, 0)


@functools.partial(
    pl.kernel,
    out_type=jax.ShapeDtypeStruct((NC, NPAD, DEGW), jnp.float32),
    mesh=_mesh,
    scratch_types=[
        pltpu.VMEM((DSTEPS, CHUNK), jnp.int32),
        pltpu.VMEM((CHUNK, DEGW), jnp.float32),
        pltpu.VMEM((CHUNK, DEGW), jnp.float32),
        pltpu.VMEM_SHARED((NPAD, DEGW), jnp.float32),
    ],
    compiler_params=pltpu.CompilerParams(use_tc_tiling_on_sc=False),
)
def _deg_pass(dst_hbm, out_hbm, dst_v, ones_v, zbuf, deg_sh):
    c = lax.axis_index("c")
    s = lax.axis_index("s")
    r0 = pl.multiple_of(s * RPT, 8)
    pltpu.sync_copy(dst_hbm.at[c, s], dst_v)
    _fill(zbuf, CHUNK, 0.0)
    _fill(ones_v, CHUNK, 1.0)
    _zero_rows(zbuf, deg_sh, r0)
    plsc.subcore_barrier()

    def step(j, carry):
        pltpu.sync_copy(ones_v, deg_sh.at[dst_v.at[j]], add=True)
        return carry
    lax.fori_loop(0, DSTEPS, step, 0)

    plsc.subcore_barrier()
    pltpu.sync_copy(deg_sh.at[pl.ds(r0, RPT)], out_hbm.at[c, pl.ds(r0, RPT)])


NBUF = 8
HALF = NBUF // 2
GROUPS = STEPS // NBUF


@functools.partial(
    pl.kernel,
    out_type=(jax.ShapeDtypeStruct((NC, NPAD, HW), jnp.float32),
              jax.ShapeDtypeStruct((NC, NPAD, HW), jnp.float32)),
    mesh=_mesh,
    scratch_types=[
        pltpu.VMEM((STEPS, CHUNK), jnp.int32),
        pltpu.VMEM((STEPS, CHUNK), jnp.int32),
        pltpu.VMEM((NBUF, CHUNK, HW), jnp.float32),
        pltpu.VMEM((CHUNK, HW), jnp.float32),
        pltpu.VMEM((CHUNK, HW), jnp.float32),
        pltpu.VMEM((CHUNK, HW), jnp.float32),
        pltpu.VMEM((HW,), jnp.float32),
        pltpu.VMEM_SHARED((NPAD, HW), jnp.float32),
        pltpu.VMEM_SHARED((NPAD, HW), jnp.float32),
        pltpu.SemaphoreType.DMA((NBUF,)),
        pltpu.SemaphoreType.DMA((NBUF,)),
    ],
    compiler_params=pltpu.CompilerParams(use_tc_tiling_on_sc=False),
)
def _gcn_core(tab_hbm, dinvw_hbm, b1_hbm, src_hbm, dst_hbm,
              acc2_hbm, h2s_hbm, src_v, dst_v, rows_v, zbuf,
              ew_a, ew_d, b1_v, acc_sh, tab_sh, gsem, ssem):
    c = lax.axis_index("c")
    s = lax.axis_index("s")
    r0 = pl.multiple_of(s * RPT, 8)
    pltpu.sync_copy(src_hbm.at[s], src_v)
    pltpu.sync_copy(dst_hbm.at[s], dst_v)
    pltpu.sync_copy(b1_hbm.at[c], b1_v)
    _fill(zbuf, CHUNK, 0.0)
    _zero_rows(zbuf, acc_sh, r0)
    pltpu.sync_copy(tab_hbm.at[c, pl.ds(r0, RPT)], tab_sh.at[pl.ds(r0, RPT)])
    plsc.subcore_barrier()

    def start_gather(j, b):
        pltpu.async_copy(tab_sh.at[src_v.at[j]], rows_v.at[b], gsem.at[b])

    def wait_gather(j, b):
        pltpu.make_async_copy(
            tab_sh.at[src_v.at[j]], rows_v.at[b], gsem.at[b]).wait()

    def start_scatter(j, b):
        pltpu.async_copy(rows_v.at[b], acc_sh.at[dst_v.at[j]], ssem.at[b],
                         add=True)

    def wait_scatter(j, b):
        pltpu.make_async_copy(
            rows_v.at[b], acc_sh.at[dst_v.at[j]], ssem.at[b]).wait()

    def edge_pass():
        for b in range(HALF):
            start_gather(b, b)

        def group(g, carry):
            for b in range(NBUF):
                j = g * NBUF + b
                wait_gather(j, b)
                start_scatter(j, b)
                bn = (b + HALF) % NBUF
                if b < HALF:
                    @pl.when(g >= 1)
                    def _():
                        wait_scatter((g - 1) * NBUF + bn, bn)
                    start_gather(j + HALF, bn)
                else:
                    @pl.when(g < GROUPS - 1)
                    def _():
                        wait_scatter(g * NBUF + bn, bn)
                        start_gather(j + HALF, bn)
            return carry
        lax.fori_loop(0, GROUPS, group, 0)

        for b in range(NBUF):
            wait_scatter(STEPS - NBUF + b, b)

    edge_pass()
    plsc.subcore_barrier()

    b1a = b1_v[pl.ds(0, 16)]
    b1b = b1_v[pl.ds(16, 16)]
    ew_t = rows_v.at[0]
    for k in range(5):
        rk = CHUNK if k < 4 else RPT - 4 * CHUNK
        base = r0 + k * CHUNK
        pltpu.sync_copy(acc_sh.at[pl.ds(base, rk)], ew_a.at[pl.ds(0, rk)])
        pltpu.sync_copy(dinvw_hbm.at[pl.ds(base, rk)], ew_d.at[pl.ds(0, rk)])
        pltpu.sync_copy(tab_sh.at[pl.ds(base, rk)], ew_t.at[pl.ds(0, rk)])

        def ew(i, carry):
            for t, b1t in ((0, b1a), (1, b1b)):
                sl = pl.ds(t * 16, 16)
                dv = ew_d[i, sl]
                agg = dv * (ew_a[i, sl] + ew_t[i, sl])
                ew_a[i, sl] = dv * jnp.maximum(agg + b1t, 0.0)
            return carry
        lax.fori_loop(0, rk, ew, 0)
        pltpu.sync_copy(ew_a.at[pl.ds(0, rk)],
                        h2s_hbm.at[c, pl.ds(base, rk)])
        pltpu.sync_copy(zbuf.at[pl.ds(0, rk)], acc_sh.at[pl.ds(base, rk)])
    pltpu.sync_copy(h2s_hbm.at[c, pl.ds(r0, RPT)], tab_sh.at[pl.ds(r0, RPT)])
    plsc.subcore_barrier()

    edge_pass()
    plsc.subcore_barrier()

    for k in range(5):
        rk = CHUNK if k < 4 else RPT - 4 * CHUNK
        base = r0 + k * CHUNK
        pltpu.sync_copy(acc_sh.at[pl.ds(base, rk)], ew_a.at[pl.ds(0, rk)])
        pltpu.sync_copy(dinvw_hbm.at[pl.ds(base, rk)], ew_d.at[pl.ds(0, rk)])
        pltpu.sync_copy(tab_sh.at[pl.ds(base, rk)], ew_t.at[pl.ds(0, rk)])

        def ew2(i, carry):
            for t in range(2):
                sl = pl.ds(t * 16, 16)
                ew_a[i, sl] = ew_d[i, sl] * (ew_a[i, sl] + ew_t[i, sl])
            return carry
        lax.fori_loop(0, rk, ew2, 0)
        pltpu.sync_copy(ew_a.at[pl.ds(0, rk)], acc2_hbm.at[c, pl.ds(base, rk)])


def _dinv(degp_ref):
    deg = degp_ref[0, 0:N, 0:1] + degp_ref[1, 0:N, 0:1] + 1.0
    return lax.rsqrt(deg)


def _tc_b_body(x_ref, w1_ref, degp_ref, out_ref, dinvw_ref):
    dinv = _dinv(degp_ref)
    h = jnp.dot(x_ref[...], w1_ref[...], preferred_element_type=jnp.float32)
    out_ref[0, 0:N, :] = dinv * h[:, 0:HW]
    out_ref[1, 0:N, :] = dinv * h[:, HW:HID]
    dinvw_ref[0:N, :] = jnp.broadcast_to(dinv, (N, HW))


def _tc_f_body(a_ref, w2_ref, b2_ref, w3_ref, b3_ref, mu_ref, lv_ref):
    aggl = a_ref[0, 0:N]
    aggr = a_ref[1, 0:N]
    mu_ref[...] = (
        jnp.dot(aggl, w2_ref[0:HW, :], preferred_element_type=jnp.float32)
        + jnp.dot(aggr, w2_ref[HW:HID, :], preferred_element_type=jnp.float32)
        + b2_ref[...])
    lv_ref[...] = (
        jnp.dot(aggl, w3_ref[0:HW, :], preferred_element_type=jnp.float32)
        + jnp.dot(aggr, w3_ref[HW:HID, :], preferred_element_type=jnp.float32)
        + b3_ref[...])


_TC_PARAMS = pltpu.CompilerParams(vmem_limit_bytes=100 * 1024 * 1024)
_tc_b = pl.pallas_call(
    _tc_b_body,
    out_shape=(jax.ShapeDtypeStruct((NC, NPAD, HW), jnp.float32),
               jax.ShapeDtypeStruct((NPAD, HW), jnp.float32)),
    compiler_params=_TC_PARAMS)
_tc_f = pl.pallas_call(
    _tc_f_body,
    out_shape=(jax.ShapeDtypeStruct((N, OUT), jnp.float32),
               jax.ShapeDtypeStruct((N, OUT), jnp.float32)),
    compiler_params=_TC_PARAMS)


@jax.jit
def kernel(x, edge_index, W1, b1, W2, b2, W3, b3):
    ei = edge_index.astype(jnp.int32)
    pad = jnp.full((2, EPAD - E), N, jnp.int32)
    eip = jnp.concatenate([ei, pad], axis=1)
    src2, dst2 = eip[0], eip[1]
    srcd = src2.reshape(NC, NS, DSTEPS, CHUNK)
    dstd = dst2.reshape(NC, NS, DSTEPS, CHUNK)
    srcp = src2.reshape(NS, STEPS, CHUNK)
    dstp = dst2.reshape(NS, STEPS, CHUNK)

    degp = _deg_pass(dstd)
    h1s, dinvw = _tc_b(x, W1, degp)
    agg2, _h2s = _gcn_core(h1s, dinvw, b1.reshape(NC, HW), srcp, dstp)
    mu, logvar = _tc_f(agg2, W2, b2.reshape(1, OUT), W3, b3.reshape(1, OUT))
    return (mu, logvar)
